# baseline (speedup 1.0000x reference)
import jax
import jax.numpy as jnp
from jax.experimental import pallas as pl
from jax.experimental.pallas import tpu as pltpu


def _layer_norm(x, eps=1e-5):
    mu = jnp.mean(x, axis=-1, keepdims=True)
    var = jnp.mean((x - mu) ** 2, axis=-1, keepdims=True)
    return (x - mu) / jnp.sqrt(var + eps)


def _gatv2(x, edge_index, edge_attr, p, N):
    src = edge_index[0]
    dst = edge_index[1]
    xl = x @ p['Wl'] + p['bl']
    xr = x @ p['Wr'] + p['br']
    m = xl[src] + xr[dst] + edge_attr @ p['We']
    a = jax.nn.leaky_relu(m, 0.2)
    alpha = a @ p['att']
    amax = jax.ops.segment_max(alpha, dst, num_segments=N)
    amax = jnp.where(jnp.isfinite(amax), amax, 0.0)
    ex = jnp.exp(alpha - amax[dst])
    den = jax.ops.segment_sum(ex, dst, num_segments=N)
    w = ex / (den[dst] + 1e-16)
    out = jax.ops.segment_sum(xl[src] * w[:, None], dst, num_segments=N)
    return out + p['bias']


def _mlp_kernel(z_ref, w1_ref, b1_ref, w2_ref, b2_ref, w3_ref, b3_ref, o_ref):
    z = z_ref[...]
    mu = jnp.mean(z, axis=-1, keepdims=True)
    var = jnp.mean((z - mu) ** 2, axis=-1, keepdims=True)
    z = (z - mu) / jnp.sqrt(var + 1e-5)
    h1 = jnp.maximum(jnp.dot(z, w1_ref[...], preferred_element_type=jnp.float32) + b1_ref[...], 0.0)
    h2 = jnp.maximum(jnp.dot(h1, w2_ref[...], preferred_element_type=jnp.float32) + b2_ref[...], 0.0)
    o_ref[...] = jnp.dot(h2, w3_ref[...], preferred_element_type=jnp.float32) + b3_ref[...]


def kernel(x, type_id, edge_index_jetjet, edge_attr_jetjet, edge_index_muonjet, edge_attr_muonjet, batch, u, params):
    N = x.shape[0]
    B = u.shape[0]
    jet = type_id == 0
    muon = type_id == 1
    hj = jax.nn.relu(x @ params['Wj'] + params['bj'])
    hm = jax.nn.relu(x @ params['Wm'] + params['bm'])
    h = jnp.where(jet[:, None], hj, 0.0) + jnp.where(muon[:, None], hm, 0.0)
    h = _gatv2(h, edge_index_jetjet, edge_attr_jetjet, params['jj1'], N) + \
        _gatv2(h, edge_index_muonjet, edge_attr_muonjet, params['mj1'], N)
    h = jax.nn.relu(_layer_norm(h))
    h = _gatv2(h, edge_index_jetjet, edge_attr_jetjet, params['jj2'], N) + \
        _gatv2(h, edge_index_muonjet, edge_attr_muonjet, params['mj2'], N)
    h = jax.nn.relu(_layer_norm(h))
    jm = jet.astype(h.dtype)
    mm = muon.astype(h.dtype)
    cj = jnp.maximum(jax.ops.segment_sum(jm, batch, num_segments=B), 1.0)
    cm = jnp.maximum(jax.ops.segment_sum(mm, batch, num_segments=B), 1.0)
    jet_mean = jax.ops.segment_sum(h * jm[:, None], batch, num_segments=B) / cj[:, None]
    muon_mean = jax.ops.segment_sum(h * mm[:, None], batch, num_segments=B) / cm[:, None]
    neg = jnp.full_like(h, -1e30)
    jet_max = jax.ops.segment_max(jnp.where(jet[:, None], h, neg), batch, num_segments=B)
    muon_max = jax.ops.segment_max(jnp.where(muon[:, None], h, neg), batch, num_segments=B)
    z = jnp.concatenate([muon_mean, jet_mean, jet_max, muon_max, u], axis=1)
    out = pl.pallas_call(
        _mlp_kernel,
        out_shape=jax.ShapeDtypeStruct((B, 1), jnp.float32),
    )(z, params['W1'], params['b1'], params['W2'], params['b2'], params['W3'], params['b3'])
    return out


# speedup vs baseline: 3.7547x; 3.7547x over previous
import functools

import jax
import jax.numpy as jnp
from jax import lax
from jax.experimental import pallas as pl
from jax.experimental.pallas import tpu as pltpu
from jax.experimental.pallas import tpu_sc as plsc

N_PAD = 10240
CD = 128
NEG = -3.0e38


def _enc_kernel(x_ref, t_ref, wj_ref, bj_ref, wm_ref, bm_ref, h_ref):
    x = x_ref[...]
    t = t_ref[...]
    hj = jnp.maximum(jnp.dot(x, wj_ref[...], preferred_element_type=jnp.float32) + bj_ref[...], 0.0)
    hm = jnp.maximum(jnp.dot(x, wm_ref[...], preferred_element_type=jnp.float32) + bm_ref[...], 0.0)
    h_ref[...] = jnp.where(t == 0, hj, 0.0) + jnp.where(t == 1, hm, 0.0)


def _encode(x_pad, t_pad, params):
    return pl.pallas_call(
        _enc_kernel,
        out_shape=jax.ShapeDtypeStruct((N_PAD, CD), jnp.float32),
    )(x_pad, t_pad, params['Wj'].astype(jnp.float32), params['bj'][None, :],
      params['Wm'], params['bm'][None, :])


def _lin_kernel(h_ref, w_ref, b_ref, o1, o2, o3, o4):
    z = jnp.dot(h_ref[...], w_ref[...], preferred_element_type=jnp.float32) + b_ref[...]
    o1[...] = z[:, 0:128]
    o2[...] = z[:, 128:256]
    o3[...] = z[:, 256:384]
    o4[...] = z[:, 384:512]


def _lin(h, pjj, pmj):
    wcat = jnp.concatenate([pjj['Wl'], pjj['Wr'], pmj['Wl'], pmj['Wr']], axis=1)
    bcat = jnp.concatenate([pjj['bl'], pjj['br'], pmj['bl'], pmj['br']])[None, :]
    RB = N_PAD // 8
    outs = pl.pallas_call(
        _lin_kernel,
        grid=(8,),
        in_specs=[pl.BlockSpec((RB, CD), lambda i: (i, 0)),
                  pl.BlockSpec((CD, 4 * CD), lambda i: (0, 0)),
                  pl.BlockSpec((1, 4 * CD), lambda i: (0, 0))],
        out_specs=[pl.BlockSpec((RB, CD), lambda i: (i, 0))] * 4,
        out_shape=[jax.ShapeDtypeStruct((N_PAD, CD), jnp.float32)] * 4,
    )(h, wcat, bcat)
    return outs


def _comb_kernel(wsj_ref, dnj_ref, amj_ref, wsm_ref, dnm_ref, amm_ref,
                 bj_ref, bm_ref, h_ref):
    def contrib(ws_ref, dn_ref, am_ref, b_ref):
        m0 = am_ref[0]
        m1 = am_ref[1]
        m = jnp.maximum(m0, m1)
        c0 = jnp.exp(m0 - m)
        c1 = jnp.exp(m1 - m)
        den = dn_ref[0] * c0 + dn_ref[1] * c1
        ws = ws_ref[0] * c0 + ws_ref[1] * c1
        return ws / (den + 1e-16) + b_ref[...]

    h = contrib(wsj_ref, dnj_ref, amj_ref, bj_ref) + contrib(wsm_ref, dnm_ref, amm_ref, bm_ref)
    mu = jnp.mean(h, axis=-1, keepdims=True)
    var = jnp.mean((h - mu) ** 2, axis=-1, keepdims=True)
    h_ref[...] = jnp.maximum((h - mu) / jnp.sqrt(var + 1e-5), 0.0)


def _combine(msg_jj, msg_mj, bias_jj, bias_mj):
    (wsj, dnj, amj) = msg_jj
    (wsm, dnm, amm) = msg_mj
    RB = N_PAD // 8
    big = pl.BlockSpec((2, RB, CD), lambda i: (0, i, 0))
    sml = pl.BlockSpec((2, RB, 1), lambda i: (0, i, 0))
    bias = pl.BlockSpec((1, CD), lambda i: (0, 0))
    return pl.pallas_call(
        _comb_kernel,
        grid=(8,),
        in_specs=[big, sml, sml, big, sml, sml, bias, bias],
        out_specs=pl.BlockSpec((RB, CD), lambda i: (i, 0)),
        out_shape=jax.ShapeDtypeStruct((N_PAD, CD), jnp.float32),
    )(wsj, dnj[:, :, None], amj[:, :, None],
      wsm, dnm[:, :, None], amm[:, :, None],
      bias_jj[None, :], bias_mj[None, :])


def _mlp_kernel(js_ref, jm_ref, ms_ref, mm_ref, jc_ref, mc_ref, u_ref,
                w1_ref, b1_ref, w2_ref, b2_ref, w3_ref, b3_ref, o_ref):
    jsum = js_ref[0] + js_ref[1]
    msum = ms_ref[0] + ms_ref[1]
    jmax = jnp.maximum(jm_ref[0], jm_ref[1])
    mmax = jnp.maximum(mm_ref[0], mm_ref[1])
    cj = jnp.maximum(jc_ref[0] + jc_ref[1], 1.0)
    cm = jnp.maximum(mc_ref[0] + mc_ref[1], 1.0)
    jmean = jsum / cj
    mmean = msum / cm
    u = u_ref[...]
    pieces = [mmean, jmean, jmax, mmax, u]
    tot = 4 * CD + 32
    s1 = sum(jnp.sum(p, axis=-1, keepdims=True) for p in pieces)
    s2 = sum(jnp.sum(p * p, axis=-1, keepdims=True) for p in pieces)
    mu = s1 / tot
    var = s2 / tot - mu * mu
    inv = 1.0 / jnp.sqrt(var + 1e-5)
    w1 = w1_ref[...]
    h1 = b1_ref[...]
    for i, p in enumerate(pieces):
        lo = i * CD
        hi = lo + (CD if i < 4 else 32)
        h1 = h1 + jnp.dot((p - mu) * inv, w1[lo:hi, :], preferred_element_type=jnp.float32)
    h1 = jnp.maximum(h1, 0.0)
    h2 = jnp.maximum(jnp.dot(h1, w2_ref[...], preferred_element_type=jnp.float32) + b2_ref[...], 0.0)
    o_ref[...] = jnp.dot(h2, w3_ref[...], preferred_element_type=jnp.float32) + b3_ref[...]


def _head(pool, u, params):
    (jsum, jmax, msum, mmax, jcnt, mcnt) = pool
    B = u.shape[0]
    return pl.pallas_call(
        _mlp_kernel,
        out_shape=jax.ShapeDtypeStruct((B, 1), jnp.float32),
    )(jsum, jmax, msum, mmax, jcnt[:, :, None], mcnt[:, :, None], u,
      params['W1'], params['b1'][None, :], params['W2'], params['b2'][None, :],
      params['W3'], params['b3'][None, :])


def _make_gat_p1(Ep, ED):
    EPT = Ep // 32
    NB = EPT // 128
    SL = N_PAD // 16
    mesh = plsc.VectorSubcoreMesh(core_axis_name="c", subcore_axis_name="s",
                                  num_cores=2, num_subcores=16)

    @functools.partial(
        pl.kernel,
        out_type=[jax.ShapeDtypeStruct((Ep,), jnp.float32),
                  jax.ShapeDtypeStruct((2, N_PAD), jnp.float32)],
        mesh=mesh,
        compiler_params=pltpu.CompilerParams(needs_layout_passes=False),
        scratch_types=[
            pltpu.VMEM((128,), jnp.int32),
            pltpu.VMEM((128,), jnp.int32),
            pltpu.VMEM((ED, 128), jnp.float32),
            pltpu.VMEM((128, CD), jnp.float32),
            pltpu.VMEM((128, CD), jnp.float32),
            pltpu.VMEM((128,), jnp.float32),
            pltpu.VMEM((N_PAD,), jnp.float32),
            pltpu.VMEM((ED, CD), jnp.float32),
            pltpu.VMEM((1, CD), jnp.float32),
            pltpu.VMEM((16, SL), jnp.float32),
            pltpu.VMEM((SL,), jnp.float32),
            pltpu.SemaphoreType.DMA,
            pltpu.VMEM_SHARED((16, N_PAD), jnp.float32),
        ],
    )
    def gat_p1(xl_hbm, xr_hbm, ei_hbm, eaT_hbm, we_hbm, att_hbm,
               alpha_out, amax_out,
               src_i, dst_i, ea_b, xl_b, xr_b, al_b, amax_t,
               we_b, att_b, mrg_b, sl_b, sem, pub_s):
        cid = lax.axis_index("c")
        sid = lax.axis_index("s")
        base = (cid * 16 + sid) * EPT

        pltpu.sync_copy(we_hbm, we_b)
        pltpu.sync_copy(att_hbm, att_b)

        def init_body(i, _):
            amax_t[pl.ds(i * 16, 16)] = jnp.full((16,), NEG, jnp.float32)
            return 0
        lax.fori_loop(0, N_PAD // 16, init_body, 0, unroll=4)

        WE = [[we_b[k, pl.ds(c * 16, 16)] for c in range(8)] for k in range(ED)]
        ATT = [att_b[0, pl.ds(c * 16, 16)] for c in range(8)]
        iota16 = lax.iota(jnp.int32, 16)

        def loop1(j, _):
            eb = base + j * 128
            pltpu.sync_copy(ei_hbm.at[0, pl.ds(eb, 128)], src_i)
            pltpu.sync_copy(ei_hbm.at[1, pl.ds(eb, 128)], dst_i)
            pltpu.sync_copy(eaT_hbm.at[:, pl.ds(eb, 128)], ea_b)
            pltpu.async_copy(xl_hbm.at[src_i], xl_b, sem).wait()
            pltpu.async_copy(xr_hbm.at[dst_i], xr_b, sem).wait()

            def grp1(v, _):
                gsl = pl.ds(v * 16, 16)
                eak = [ea_b[k, gsl] for k in range(ED)]
                alpha16 = jnp.zeros((16,), jnp.float32)
                for el in range(16):
                    acc = jnp.zeros((16,), jnp.float32)
                    for c in range(8):
                        sl = pl.ds(c * 16, 16)
                        row = v * 16 + el
                        m = xl_b[row, sl] + xr_b[row, sl]
                        for k in range(ED):
                            m = m + eak[k][el] * WE[k][c]
                        a = jnp.where(m >= 0, m, 0.2 * m)
                        acc = acc + a * ATT[c]
                    al = jnp.sum(acc)
                    alpha16 = jnp.where(iota16 == el, al, alpha16)
                al_b[gsl] = alpha16
                d16 = dst_i[gsl]
                got = plsc.load_gather(amax_t, [d16])
                need = alpha16 > got

                def wbody(nd):
                    plsc.store_scatter(amax_t, [d16], alpha16, mask=nd)
                    g2 = plsc.load_gather(amax_t, [d16])
                    return alpha16 > g2
                lax.while_loop(lambda nd: jnp.any(nd), wbody, need)
                return 0
            lax.fori_loop(0, 8, grp1, 0)
            pltpu.sync_copy(al_b, alpha_out.at[pl.ds(eb, 128)])
            return 0
        lax.fori_loop(0, NB, loop1, 0)

        pltpu.sync_copy(amax_t, pub_s.at[sid])
        plsc.subcore_barrier()
        pltpu.sync_copy(pub_s.at[:, pl.ds(sid * SL, SL)], mrg_b)

        def mmax(v, _):
            sl = pl.ds(v * 16, 16)
            m = mrg_b[0, sl]
            for t in range(1, 16):
                m = jnp.maximum(m, mrg_b[t, sl])
            sl_b[sl] = m
            return 0
        lax.fori_loop(0, SL // 16, mmax, 0, unroll=4)
        pltpu.sync_copy(sl_b, amax_out.at[cid, pl.ds(sid * SL, SL)])

    return gat_p1


def _make_gat_p2(Ep):
    EPT = Ep // 32
    NB = EPT // 128
    SL = N_PAD // 16
    mesh = plsc.VectorSubcoreMesh(core_axis_name="c", subcore_axis_name="s",
                                  num_cores=2, num_subcores=16)

    @functools.partial(
        pl.kernel,
        out_type=[jax.ShapeDtypeStruct((2, N_PAD, CD), jnp.float32),
                  jax.ShapeDtypeStruct((2, N_PAD), jnp.float32)],
        mesh=mesh,
        compiler_params=pltpu.CompilerParams(needs_layout_passes=False),
        scratch_types=[
            pltpu.VMEM((128,), jnp.int32),
            pltpu.VMEM((128,), jnp.int32),
            pltpu.VMEM((128, CD), jnp.float32),
            pltpu.VMEM((128, CD), jnp.float32),
            pltpu.VMEM((128,), jnp.float32),
            pltpu.VMEM((128,), jnp.float32),
            pltpu.VMEM((N_PAD,), jnp.float32),
            pltpu.VMEM((SL,), jnp.float32),
            pltpu.SemaphoreType.DMA,
            pltpu.VMEM_SHARED((N_PAD, CD), jnp.float32),
            pltpu.VMEM_SHARED((N_PAD,), jnp.float32),
        ],
    )
    def gat_p2(xl_hbm, ei_hbm, alpha_hbm, amax_hbm,
               wsum_out, den_out,
               src_i, dst_i, xl_b, wr_b, al_b, ex_b, amax_t, sl_b, sem,
               wsum_s, den_s):
        cid = lax.axis_index("c")
        sid = lax.axis_index("s")
        base = (cid * 16 + sid) * EPT

        pltpu.sync_copy(amax_hbm.at[cid], amax_t)

        def zrow(e, _):
            for c in range(8):
                wr_b[e, pl.ds(c * 16, 16)] = jnp.zeros((16,), jnp.float32)
            return 0
        lax.fori_loop(0, 128, zrow, 0)

        def zsl(i, _):
            sl_b[pl.ds(i * 16, 16)] = jnp.zeros((16,), jnp.float32)
            return 0
        lax.fori_loop(0, SL // 16, zsl, 0, unroll=4)
        for k in range(SL // 128):
            pltpu.sync_copy(wr_b, wsum_s.at[pl.ds(sid * SL + k * 128, 128)])
        pltpu.sync_copy(sl_b, den_s.at[pl.ds(sid * SL, SL)])
        plsc.subcore_barrier()

        def loop2(j, _):
            eb = base + j * 128
            pltpu.sync_copy(ei_hbm.at[0, pl.ds(eb, 128)], src_i)
            pltpu.sync_copy(ei_hbm.at[1, pl.ds(eb, 128)], dst_i)
            pltpu.sync_copy(alpha_hbm.at[pl.ds(eb, 128)], al_b)
            pltpu.async_copy(xl_hbm.at[src_i], xl_b, sem).wait()

            def grp2(v, _):
                gsl = pl.ds(v * 16, 16)
                d16 = dst_i[gsl]
                am = plsc.load_gather(amax_t, [d16])
                ex16 = jnp.exp(al_b[gsl] - am)
                ex_b[gsl] = ex16
                for el in range(16):
                    row = v * 16 + el
                    exe = jnp.full((16,), ex16[el], jnp.float32)
                    for c in range(8):
                        sl = pl.ds(c * 16, 16)
                        wr_b[row, sl] = xl_b[row, sl] * exe
                return 0
            lax.fori_loop(0, 8, grp2, 0)
            pltpu.sync_copy(wr_b, wsum_s.at[dst_i], add=True)
            pltpu.sync_copy(ex_b, den_s.at[dst_i], add=True)
            return 0
        lax.fori_loop(0, NB, loop2, 0)

        plsc.subcore_barrier()
        pltpu.sync_copy(wsum_s.at[pl.ds(sid * SL, SL)],
                        wsum_out.at[cid, pl.ds(sid * SL, SL)])
        pltpu.sync_copy(den_s.at[pl.ds(sid * SL, SL)],
                        den_out.at[cid, pl.ds(sid * SL, SL)])

    return gat_p2


def _gat_messages_sc(xl, xr, ei, eaT, p):
    Ep = ei.shape[1]
    ED = eaT.shape[0]
    alpha, amax = _make_gat_p1(Ep, ED)(xl, xr, ei, eaT, p['We'], p['att'][None, :])
    wsum, den = _make_gat_p2(Ep)(xl, ei, alpha, amax)
    return wsum, den, amax


def _gat_messages_jax(xl, xr, src, dst, ea, p):
    E = src.shape[0]
    m = xl[src] + xr[dst] + ea @ p['We']
    a = jnp.where(m >= 0, m, 0.2 * m)
    alpha = a @ p['att']
    amax = jax.ops.segment_max(alpha, dst, num_segments=N_PAD)
    amax = jnp.where(jnp.isfinite(amax), amax, NEG)
    ex = jnp.exp(alpha - amax[dst])
    den = jax.ops.segment_sum(ex, dst, num_segments=N_PAD)
    ws = jax.ops.segment_sum(xl[src] * ex[:, None], dst, num_segments=N_PAD)
    wsum = jnp.stack([ws, jnp.zeros_like(ws)])
    den2 = jnp.stack([den, jnp.zeros_like(den)])
    amax2 = jnp.stack([amax, jnp.full_like(amax, NEG)])
    return wsum, den2, amax2


def _pool_jax(h, t_pad, batch_pad):
    B = 64
    jet = (t_pad == 0).astype(jnp.float32)
    muon = (t_pad == 1).astype(jnp.float32)
    jcnt = jax.ops.segment_sum(jet, batch_pad, num_segments=B)
    mcnt = jax.ops.segment_sum(muon, batch_pad, num_segments=B)
    jsum = jax.ops.segment_sum(h * jet[:, None], batch_pad, num_segments=B)
    msum = jax.ops.segment_sum(h * muon[:, None], batch_pad, num_segments=B)
    neg = jnp.full_like(h, -1e30)
    jmax = jax.ops.segment_max(jnp.where(jet[:, None] > 0, h, neg), batch_pad, num_segments=B)
    mmax = jax.ops.segment_max(jnp.where(muon[:, None] > 0, h, neg), batch_pad, num_segments=B)
    z = jnp.zeros((B, CD), jnp.float32)
    zc = jnp.zeros((B,), jnp.float32)
    return (jnp.stack([jsum, jnp.zeros_like(jsum)]),
            jnp.stack([jmax, jnp.full_like(jmax, -1e30)]),
            jnp.stack([msum, jnp.zeros_like(msum)]),
            jnp.stack([mmax, jnp.full_like(mmax, -1e30)]),
            jnp.stack([jcnt, zc]), jnp.stack([mcnt, zc]))


def kernel(x, type_id, edge_index_jetjet, edge_attr_jetjet, edge_index_muonjet,
           edge_attr_muonjet, batch, u, params):
    N = x.shape[0]
    pad = N_PAD - N
    x_pad = jnp.pad(x, ((0, pad), (0, 0)))
    t_pad = jnp.pad(type_id.astype(jnp.int32), (0, pad), constant_values=2)[:, None]
    batch_pad = jnp.pad(batch.astype(jnp.int32), (0, pad), constant_values=63)

    def prep_edges(ei, ea, block=4096):
        E = ei.shape[1]
        Ep = ((E + block - 1) // block) * block
        ei_pad = jnp.pad(ei.astype(jnp.int32), ((0, 0), (0, Ep - E)), constant_values=N + 8)
        eaT = jnp.pad(ea, ((0, Ep - E), (0, 0))).T
        return ei_pad, eaT

    ei_jj, eaT_jj = prep_edges(edge_index_jetjet, edge_attr_jetjet)
    ei_mj, eaT_mj = prep_edges(edge_index_muonjet, edge_attr_muonjet)

    h = _encode(x_pad, t_pad, params)

    for lname_jj, lname_mj in (('jj1', 'mj1'), ('jj2', 'mj2')):
        pjj = params[lname_jj]
        pmj = params[lname_mj]
        xl_jj, xr_jj, xl_mj, xr_mj = _lin(h, pjj, pmj)
        msg_jj = _gat_messages_sc(xl_jj, xr_jj, ei_jj, eaT_jj, pjj)
        msg_mj = _gat_messages_sc(xl_mj, xr_mj, ei_mj, eaT_mj, pmj)
        h = _combine(msg_jj, msg_mj, pjj['bias'], pmj['bias'])

    pool = _pool_jax(h, t_pad[:, 0], batch_pad)
    return _head(pool, u, params)


# speedup vs baseline: 3.7871x; 1.0086x over previous
import functools

import jax
import jax.numpy as jnp
from jax import lax
from jax.experimental import pallas as pl
from jax.experimental.pallas import tpu as pltpu
from jax.experimental.pallas import tpu_sc as plsc

N_PAD = 10240
CD = 128
NEG = -3.0e38


def _enc_kernel(x_ref, t_ref, wj_ref, bj_ref, wm_ref, bm_ref, h_ref):
    x = x_ref[...]
    t = t_ref[...]
    hj = jnp.maximum(jnp.dot(x, wj_ref[...], preferred_element_type=jnp.float32) + bj_ref[...], 0.0)
    hm = jnp.maximum(jnp.dot(x, wm_ref[...], preferred_element_type=jnp.float32) + bm_ref[...], 0.0)
    h_ref[...] = jnp.where(t == 0, hj, 0.0) + jnp.where(t == 1, hm, 0.0)


def _encode(x_pad, t_pad, params):
    return pl.pallas_call(
        _enc_kernel,
        out_shape=jax.ShapeDtypeStruct((N_PAD, CD), jnp.float32),
    )(x_pad, t_pad, params['Wj'].astype(jnp.float32), params['bj'][None, :],
      params['Wm'], params['bm'][None, :])


def _lin_kernel(h_ref, w_ref, b_ref, o1, o2, o3, o4):
    z = jnp.dot(h_ref[...], w_ref[...], preferred_element_type=jnp.float32) + b_ref[...]
    o1[...] = z[:, 0:128]
    o2[...] = z[:, 128:256]
    o3[...] = z[:, 256:384]
    o4[...] = z[:, 384:512]


def _lin(h, pjj, pmj):
    wcat = jnp.concatenate([pjj['Wl'], pjj['Wr'], pmj['Wl'], pmj['Wr']], axis=1)
    bcat = jnp.concatenate([pjj['bl'], pjj['br'], pmj['bl'], pmj['br']])[None, :]
    RB = N_PAD // 8
    outs = pl.pallas_call(
        _lin_kernel,
        grid=(8,),
        in_specs=[pl.BlockSpec((RB, CD), lambda i: (i, 0)),
                  pl.BlockSpec((CD, 4 * CD), lambda i: (0, 0)),
                  pl.BlockSpec((1, 4 * CD), lambda i: (0, 0))],
        out_specs=[pl.BlockSpec((RB, CD), lambda i: (i, 0))] * 4,
        out_shape=[jax.ShapeDtypeStruct((N_PAD, CD), jnp.float32)] * 4,
    )(h, wcat, bcat)
    return outs


def _comb_kernel(wsj_ref, dnj_ref, amj_ref, wsm_ref, dnm_ref, amm_ref,
                 bj_ref, bm_ref, h_ref):
    def contrib(ws_ref, dn_ref, am_ref, b_ref):
        m0 = am_ref[0]
        m1 = am_ref[1]
        m = jnp.maximum(m0, m1)
        c0 = jnp.exp(m0 - m)
        c1 = jnp.exp(m1 - m)
        den = dn_ref[0] * c0 + dn_ref[1] * c1
        ws = ws_ref[0] * c0 + ws_ref[1] * c1
        return ws / (den + 1e-16) + b_ref[...]

    h = contrib(wsj_ref, dnj_ref, amj_ref, bj_ref) + contrib(wsm_ref, dnm_ref, amm_ref, bm_ref)
    mu = jnp.mean(h, axis=-1, keepdims=True)
    var = jnp.mean((h - mu) ** 2, axis=-1, keepdims=True)
    h_ref[...] = jnp.maximum((h - mu) / jnp.sqrt(var + 1e-5), 0.0)


def _combine(msg_jj, msg_mj, bias_jj, bias_mj):
    (wsj, dnj, amj) = msg_jj
    (wsm, dnm, amm) = msg_mj
    RB = N_PAD // 8
    big = pl.BlockSpec((2, RB, CD), lambda i: (0, i, 0))
    sml = pl.BlockSpec((2, RB, 1), lambda i: (0, i, 0))
    bias = pl.BlockSpec((1, CD), lambda i: (0, 0))
    return pl.pallas_call(
        _comb_kernel,
        grid=(8,),
        in_specs=[big, sml, sml, big, sml, sml, bias, bias],
        out_specs=pl.BlockSpec((RB, CD), lambda i: (i, 0)),
        out_shape=jax.ShapeDtypeStruct((N_PAD, CD), jnp.float32),
    )(wsj, dnj[:, :, None], amj[:, :, None],
      wsm, dnm[:, :, None], amm[:, :, None],
      bias_jj[None, :], bias_mj[None, :])


def _mlp_kernel(js_ref, jm_ref, ms_ref, mm_ref, jc_ref, mc_ref, u_ref,
                w1_ref, b1_ref, w2_ref, b2_ref, w3_ref, b3_ref, o_ref):
    jsum = js_ref[0] + js_ref[1]
    msum = ms_ref[0] + ms_ref[1]
    jmax = jnp.maximum(jm_ref[0], jm_ref[1])
    mmax = jnp.maximum(mm_ref[0], mm_ref[1])
    cj = jnp.maximum(jnp.sum(jc_ref[...], axis=0), 1.0)
    cm = jnp.maximum(jnp.sum(mc_ref[...], axis=0), 1.0)
    jmean = jsum / cj
    mmean = msum / cm
    u = u_ref[...]
    pieces = [mmean, jmean, jmax, mmax, u]
    tot = 4 * CD + 32
    s1 = sum(jnp.sum(p, axis=-1, keepdims=True) for p in pieces)
    s2 = sum(jnp.sum(p * p, axis=-1, keepdims=True) for p in pieces)
    mu = s1 / tot
    var = s2 / tot - mu * mu
    inv = 1.0 / jnp.sqrt(var + 1e-5)
    w1 = w1_ref[...]
    h1 = b1_ref[...]
    for i, p in enumerate(pieces):
        lo = i * CD
        hi = lo + (CD if i < 4 else 32)
        h1 = h1 + jnp.dot((p - mu) * inv, w1[lo:hi, :], preferred_element_type=jnp.float32)
    h1 = jnp.maximum(h1, 0.0)
    h2 = jnp.maximum(jnp.dot(h1, w2_ref[...], preferred_element_type=jnp.float32) + b2_ref[...], 0.0)
    o_ref[...] = jnp.dot(h2, w3_ref[...], preferred_element_type=jnp.float32) + b3_ref[...]


def _head(pool, u, params):
    (jsum, jmax, msum, mmax, jcnt, mcnt) = pool
    B = u.shape[0]
    return pl.pallas_call(
        _mlp_kernel,
        out_shape=jax.ShapeDtypeStruct((B, 1), jnp.float32),
    )(jsum, jmax, msum, mmax, jcnt, mcnt, u,
      params['W1'], params['b1'][None, :], params['W2'], params['b2'][None, :],
      params['W3'], params['b3'][None, :])


def _make_gat_p1(Ep, ED):
    EPT = Ep // 32
    NB = EPT // 128
    SL = N_PAD // 16
    mesh = plsc.VectorSubcoreMesh(core_axis_name="c", subcore_axis_name="s",
                                  num_cores=2, num_subcores=16)

    @functools.partial(
        pl.kernel,
        out_type=[jax.ShapeDtypeStruct((Ep,), jnp.float32),
                  jax.ShapeDtypeStruct((2, N_PAD), jnp.float32)],
        mesh=mesh,
        compiler_params=pltpu.CompilerParams(needs_layout_passes=False),
        scratch_types=[
            pltpu.VMEM((128,), jnp.int32),
            pltpu.VMEM((128,), jnp.int32),
            pltpu.VMEM((ED, 128), jnp.float32),
            pltpu.VMEM((128, CD), jnp.float32),
            pltpu.VMEM((128, CD), jnp.float32),
            pltpu.VMEM((128,), jnp.float32),
            pltpu.VMEM((N_PAD,), jnp.float32),
            pltpu.VMEM((ED, CD), jnp.float32),
            pltpu.VMEM((1, CD), jnp.float32),
            pltpu.VMEM((16, SL), jnp.float32),
            pltpu.VMEM((SL,), jnp.float32),
            pltpu.SemaphoreType.DMA,
            pltpu.VMEM_SHARED((16, N_PAD), jnp.float32),
        ],
    )
    def gat_p1(xl_hbm, xr_hbm, ei_hbm, eaT_hbm, we_hbm, att_hbm,
               alpha_out, amax_out,
               src_i, dst_i, ea_b, xl_b, xr_b, al_b, amax_t,
               we_b, att_b, mrg_b, sl_b, sem, pub_s):
        cid = lax.axis_index("c")
        sid = lax.axis_index("s")
        base = (cid * 16 + sid) * EPT

        pltpu.sync_copy(we_hbm, we_b)
        pltpu.sync_copy(att_hbm, att_b)

        def init_body(i, _):
            amax_t[pl.ds(i * 16, 16)] = jnp.full((16,), NEG, jnp.float32)
            return 0
        lax.fori_loop(0, N_PAD // 16, init_body, 0, unroll=4)

        WE = [[we_b[k, pl.ds(c * 16, 16)] for c in range(8)] for k in range(ED)]
        ATT = [att_b[0, pl.ds(c * 16, 16)] for c in range(8)]
        iota16 = lax.iota(jnp.int32, 16)

        def loop1(j, _):
            eb = base + j * 128
            pltpu.sync_copy(ei_hbm.at[0, pl.ds(eb, 128)], src_i)
            pltpu.sync_copy(ei_hbm.at[1, pl.ds(eb, 128)], dst_i)
            pltpu.sync_copy(eaT_hbm.at[:, pl.ds(eb, 128)], ea_b)
            pltpu.async_copy(xl_hbm.at[src_i], xl_b, sem).wait()
            pltpu.async_copy(xr_hbm.at[dst_i], xr_b, sem).wait()

            def grp1(v, _):
                gsl = pl.ds(v * 16, 16)
                eak = [ea_b[k, gsl] for k in range(ED)]
                alpha16 = jnp.zeros((16,), jnp.float32)
                for el in range(16):
                    acc = jnp.zeros((16,), jnp.float32)
                    for c in range(8):
                        sl = pl.ds(c * 16, 16)
                        row = v * 16 + el
                        m = xl_b[row, sl] + xr_b[row, sl]
                        for k in range(ED):
                            m = m + eak[k][el] * WE[k][c]
                        a = jnp.where(m >= 0, m, 0.2 * m)
                        acc = acc + a * ATT[c]
                    al = jnp.sum(acc)
                    alpha16 = jnp.where(iota16 == el, al, alpha16)
                al_b[gsl] = alpha16
                d16 = dst_i[gsl]
                got = plsc.load_gather(amax_t, [d16])
                need = alpha16 > got

                def wbody(nd):
                    plsc.store_scatter(amax_t, [d16], alpha16, mask=nd)
                    g2 = plsc.load_gather(amax_t, [d16])
                    return alpha16 > g2
                lax.while_loop(lambda nd: jnp.any(nd), wbody, need)
                return 0
            lax.fori_loop(0, 8, grp1, 0)
            pltpu.sync_copy(al_b, alpha_out.at[pl.ds(eb, 128)])
            return 0
        lax.fori_loop(0, NB, loop1, 0)

        pltpu.sync_copy(amax_t, pub_s.at[sid])
        plsc.subcore_barrier()
        pltpu.sync_copy(pub_s.at[:, pl.ds(sid * SL, SL)], mrg_b)

        def mmax(v, _):
            sl = pl.ds(v * 16, 16)
            m = mrg_b[0, sl]
            for t in range(1, 16):
                m = jnp.maximum(m, mrg_b[t, sl])
            sl_b[sl] = m
            return 0
        lax.fori_loop(0, SL // 16, mmax, 0, unroll=4)
        pltpu.sync_copy(sl_b, amax_out.at[cid, pl.ds(sid * SL, SL)])

    return gat_p1


def _make_gat_p2(Ep):
    EPT = Ep // 32
    NB = EPT // 128
    SL = N_PAD // 16
    mesh = plsc.VectorSubcoreMesh(core_axis_name="c", subcore_axis_name="s",
                                  num_cores=2, num_subcores=16)

    @functools.partial(
        pl.kernel,
        out_type=[jax.ShapeDtypeStruct((2, N_PAD, CD), jnp.float32),
                  jax.ShapeDtypeStruct((2, N_PAD), jnp.float32)],
        mesh=mesh,
        compiler_params=pltpu.CompilerParams(needs_layout_passes=False),
        scratch_types=[
            pltpu.VMEM((128,), jnp.int32),
            pltpu.VMEM((128,), jnp.int32),
            pltpu.VMEM((128, CD), jnp.float32),
            pltpu.VMEM((128, CD), jnp.float32),
            pltpu.VMEM((128,), jnp.float32),
            pltpu.VMEM((128,), jnp.float32),
            pltpu.VMEM((N_PAD,), jnp.float32),
            pltpu.VMEM((SL,), jnp.float32),
            pltpu.SemaphoreType.DMA,
            pltpu.VMEM_SHARED((N_PAD, CD), jnp.float32),
            pltpu.VMEM_SHARED((N_PAD,), jnp.float32),
        ],
    )
    def gat_p2(xl_hbm, ei_hbm, alpha_hbm, amax_hbm,
               wsum_out, den_out,
               src_i, dst_i, xl_b, wr_b, al_b, ex_b, amax_t, sl_b, sem,
               wsum_s, den_s):
        cid = lax.axis_index("c")
        sid = lax.axis_index("s")
        base = (cid * 16 + sid) * EPT

        pltpu.sync_copy(amax_hbm.at[cid], amax_t)

        def zrow(e, _):
            for c in range(8):
                wr_b[e, pl.ds(c * 16, 16)] = jnp.zeros((16,), jnp.float32)
            return 0
        lax.fori_loop(0, 128, zrow, 0)

        def zsl(i, _):
            sl_b[pl.ds(i * 16, 16)] = jnp.zeros((16,), jnp.float32)
            return 0
        lax.fori_loop(0, SL // 16, zsl, 0, unroll=4)
        for k in range(SL // 128):
            pltpu.sync_copy(wr_b, wsum_s.at[pl.ds(sid * SL + k * 128, 128)])
        pltpu.sync_copy(sl_b, den_s.at[pl.ds(sid * SL, SL)])
        plsc.subcore_barrier()

        def loop2(j, _):
            eb = base + j * 128
            pltpu.sync_copy(ei_hbm.at[0, pl.ds(eb, 128)], src_i)
            pltpu.sync_copy(ei_hbm.at[1, pl.ds(eb, 128)], dst_i)
            pltpu.sync_copy(alpha_hbm.at[pl.ds(eb, 128)], al_b)
            pltpu.async_copy(xl_hbm.at[src_i], xl_b, sem).wait()

            def grp2(v, _):
                gsl = pl.ds(v * 16, 16)
                d16 = dst_i[gsl]
                am = plsc.load_gather(amax_t, [d16])
                ex16 = jnp.exp(al_b[gsl] - am)
                ex_b[gsl] = ex16
                for el in range(16):
                    row = v * 16 + el
                    exe = jnp.full((16,), ex16[el], jnp.float32)
                    for c in range(8):
                        sl = pl.ds(c * 16, 16)
                        wr_b[row, sl] = xl_b[row, sl] * exe
                return 0
            lax.fori_loop(0, 8, grp2, 0)
            pltpu.sync_copy(wr_b, wsum_s.at[dst_i], add=True)
            pltpu.sync_copy(ex_b, den_s.at[dst_i], add=True)
            return 0
        lax.fori_loop(0, NB, loop2, 0)

        plsc.subcore_barrier()
        pltpu.sync_copy(wsum_s.at[pl.ds(sid * SL, SL)],
                        wsum_out.at[cid, pl.ds(sid * SL, SL)])
        pltpu.sync_copy(den_s.at[pl.ds(sid * SL, SL)],
                        den_out.at[cid, pl.ds(sid * SL, SL)])

    return gat_p2


def _gat_messages_sc(xl, xr, ei, eaT, p):
    Ep = ei.shape[1]
    ED = eaT.shape[0]
    alpha, amax = _make_gat_p1(Ep, ED)(xl, xr, ei, eaT, p['We'], p['att'][None, :])
    wsum, den = _make_gat_p2(Ep)(xl, ei, alpha, amax)
    return wsum, den, amax


def _make_pool():
    RPT = N_PAD // 32
    B = 64
    mesh = plsc.VectorSubcoreMesh(core_axis_name="c", subcore_axis_name="s",
                                  num_cores=2, num_subcores=16)

    @functools.partial(
        pl.kernel,
        out_type=[jax.ShapeDtypeStruct((2, B, CD), jnp.float32)] * 4 +
                 [jax.ShapeDtypeStruct((32, B, 16), jnp.float32)] * 2,
        mesh=mesh,
        compiler_params=pltpu.CompilerParams(needs_layout_passes=False),
        scratch_types=[
            pltpu.VMEM((RPT // 4, CD), jnp.float32),
            pltpu.VMEM((RPT,), jnp.int32),
            pltpu.VMEM((RPT,), jnp.int32),
            pltpu.VMEM((B, CD), jnp.float32),
            pltpu.VMEM((B, CD), jnp.float32),
            pltpu.VMEM((B, CD), jnp.float32),
            pltpu.VMEM((B, CD), jnp.float32),
            pltpu.VMEM((B, 16), jnp.float32),
            pltpu.VMEM((B, 16), jnp.float32),
            pltpu.VMEM((16, 2, CD), jnp.float32),
            pltpu.VMEM((2, CD), jnp.float32),
            pltpu.SemaphoreType.DMA,
            pltpu.VMEM_SHARED((16, B, CD), jnp.float32),
            pltpu.VMEM_SHARED((16, B, CD), jnp.float32),
            pltpu.VMEM_SHARED((16, B, CD), jnp.float32),
            pltpu.VMEM_SHARED((16, B, CD), jnp.float32),
        ],
    )
    def pool(h_hbm, t_hbm, b_hbm, js_o, jm_o, ms_o, mm_o, jc_o, mc_o,
             h_b, bt_b, tp_b, js_t, jm_t, ms_t, mm_t, jc_t, mc_t,
             mrg, osl, sem,
             js_s, jm_s, ms_s, mm_s):
        cid = lax.axis_index("c")
        sid = lax.axis_index("s")
        rb = (cid * 16 + sid) * RPT

        pltpu.sync_copy(b_hbm.at[pl.ds(rb, RPT)], bt_b)
        pltpu.sync_copy(t_hbm.at[pl.ds(rb, RPT)], tp_b)

        zero16 = jnp.zeros((16,), jnp.float32)
        neg16 = jnp.full((16,), -1e30, jnp.float32)
        iota16 = lax.iota(jnp.int32, 16)
        one0 = jnp.where(iota16 == 0, 1.0, 0.0).astype(jnp.float32)

        def init_body(i, _):
            for c in range(8):
                sl = pl.ds(c * 16, 16)
                js_t[i, sl] = zero16
                ms_t[i, sl] = zero16
                jm_t[i, sl] = neg16
                mm_t[i, sl] = neg16
            jc_t[i] = zero16
            mc_t[i] = zero16
            return 0
        lax.fori_loop(0, B, init_body, 0)

        HC = RPT // 4
        for ch in range(4):
            pltpu.sync_copy(h_hbm.at[pl.ds(rb + ch * HC, HC), :], h_b)

            def rows(g, _):
                b16 = bt_b[pl.ds(ch * HC + g * 16, 16)]
                t16 = tp_b[pl.ds(ch * HC + g * 16, 16)]
                for el in range(16):
                    b = b16[el]
                    t = t16[el]
                    row = g * 16 + el

                    @pl.when(t == 0)
                    def _():
                        for c in range(8):
                            sl = pl.ds(c * 16, 16)
                            v = h_b[row, sl]
                            js_t[b, sl] = js_t[b, sl] + v
                            jm_t[b, sl] = jnp.maximum(jm_t[b, sl], v)
                        jc_t[b] = jc_t[b] + one0

                    @pl.when(t == 1)
                    def _():
                        for c in range(8):
                            sl = pl.ds(c * 16, 16)
                            v = h_b[row, sl]
                            ms_t[b, sl] = ms_t[b, sl] + v
                            mm_t[b, sl] = jnp.maximum(mm_t[b, sl], v)
                        mc_t[b] = mc_t[b] + one0
                return 0
            lax.fori_loop(0, HC // 16, rows, 0)

        pltpu.sync_copy(js_t, js_s.at[sid])
        pltpu.sync_copy(jm_t, jm_s.at[sid])
        pltpu.sync_copy(ms_t, ms_s.at[sid])
        pltpu.sync_copy(mm_t, mm_s.at[sid])
        pltpu.sync_copy(jc_t, jc_o.at[cid * 16 + sid])
        pltpu.sync_copy(mc_t, mc_o.at[cid * 16 + sid])
        plsc.subcore_barrier()

        for tbl_s, out_ref, is_max in ((js_s, js_o, False), (jm_s, jm_o, True),
                                       (ms_s, ms_o, False), (mm_s, mm_o, True)):
            for half in range(2):
                r2 = pl.ds(sid * 4 + half * 2, 2)
                pltpu.sync_copy(tbl_s.at[:, r2, :], mrg)
                for r in range(2):
                    for c in range(8):
                        sl = pl.ds(c * 16, 16)
                        acc = mrg[0, r, sl]
                        for t in range(1, 16):
                            v = mrg[t, r, sl]
                            acc = jnp.maximum(acc, v) if is_max else acc + v
                        osl[r, sl] = acc
                pltpu.sync_copy(osl, out_ref.at[cid, r2, :])

    return pool


def _pool_sc(h, t_pad, batch_pad):
    js, jm, ms, mm, jc, mc = _make_pool()(h, t_pad, batch_pad)
    return js, jm, ms, mm, jc[:, :, 0:1], mc[:, :, 0:1]


def _gat_messages_jax(xl, xr, src, dst, ea, p):
    E = src.shape[0]
    m = xl[src] + xr[dst] + ea @ p['We']
    a = jnp.where(m >= 0, m, 0.2 * m)
    alpha = a @ p['att']
    amax = jax.ops.segment_max(alpha, dst, num_segments=N_PAD)
    amax = jnp.where(jnp.isfinite(amax), amax, NEG)
    ex = jnp.exp(alpha - amax[dst])
    den = jax.ops.segment_sum(ex, dst, num_segments=N_PAD)
    ws = jax.ops.segment_sum(xl[src] * ex[:, None], dst, num_segments=N_PAD)
    wsum = jnp.stack([ws, jnp.zeros_like(ws)])
    den2 = jnp.stack([den, jnp.zeros_like(den)])
    amax2 = jnp.stack([amax, jnp.full_like(amax, NEG)])
    return wsum, den2, amax2


def _pool_jax(h, t_pad, batch_pad):
    B = 64
    jet = (t_pad == 0).astype(jnp.float32)
    muon = (t_pad == 1).astype(jnp.float32)
    jcnt = jax.ops.segment_sum(jet, batch_pad, num_segments=B)
    mcnt = jax.ops.segment_sum(muon, batch_pad, num_segments=B)
    jsum = jax.ops.segment_sum(h * jet[:, None], batch_pad, num_segments=B)
    msum = jax.ops.segment_sum(h * muon[:, None], batch_pad, num_segments=B)
    neg = jnp.full_like(h, -1e30)
    jmax = jax.ops.segment_max(jnp.where(jet[:, None] > 0, h, neg), batch_pad, num_segments=B)
    mmax = jax.ops.segment_max(jnp.where(muon[:, None] > 0, h, neg), batch_pad, num_segments=B)
    z = jnp.zeros((B, CD), jnp.float32)
    zc = jnp.zeros((B,), jnp.float32)
    return (jnp.stack([jsum, jnp.zeros_like(jsum)]),
            jnp.stack([jmax, jnp.full_like(jmax, -1e30)]),
            jnp.stack([msum, jnp.zeros_like(msum)]),
            jnp.stack([mmax, jnp.full_like(mmax, -1e30)]),
            jnp.stack([jcnt, zc]), jnp.stack([mcnt, zc]))


def kernel(x, type_id, edge_index_jetjet, edge_attr_jetjet, edge_index_muonjet,
           edge_attr_muonjet, batch, u, params):
    N = x.shape[0]
    pad = N_PAD - N
    x_pad = jnp.pad(x, ((0, pad), (0, 0)))
    t_pad = jnp.pad(type_id.astype(jnp.int32), (0, pad), constant_values=2)[:, None]
    batch_pad = jnp.pad(batch.astype(jnp.int32), (0, pad), constant_values=63)

    def prep_edges(ei, ea, block=4096):
        E = ei.shape[1]
        Ep = ((E + block - 1) // block) * block
        ei_pad = jnp.pad(ei.astype(jnp.int32), ((0, 0), (0, Ep - E)), constant_values=N + 8)
        eaT = jnp.pad(ea, ((0, Ep - E), (0, 0))).T
        return ei_pad, eaT

    ei_jj, eaT_jj = prep_edges(edge_index_jetjet, edge_attr_jetjet)
    ei_mj, eaT_mj = prep_edges(edge_index_muonjet, edge_attr_muonjet)

    h = _encode(x_pad, t_pad, params)

    for lname_jj, lname_mj in (('jj1', 'mj1'), ('jj2', 'mj2')):
        pjj = params[lname_jj]
        pmj = params[lname_mj]
        xl_jj, xr_jj, xl_mj, xr_mj = _lin(h, pjj, pmj)
        msg_jj = _gat_messages_sc(xl_jj, xr_jj, ei_jj, eaT_jj, pjj)
        msg_mj = _gat_messages_sc(xl_mj, xr_mj, ei_mj, eaT_mj, pmj)
        h = _combine(msg_jj, msg_mj, pjj['bias'], pmj['bias'])

    pool = _pool_sc(h, t_pad[:, 0], batch_pad)
    return _head(pool, u, params)
